# Initial kernel scaffold; baseline (speedup 1.0000x reference)
import jax
import jax.numpy as jnp
from jax.experimental import pallas as pl


def kernel(input_variable, table):
    raise NotImplementedError("write your pallas kernel here")


# speedup vs baseline: 1.4770x; 1.4770x over previous
import functools

import jax
import jax.numpy as jnp
from jax import lax
from jax.experimental import pallas as pl
from jax.experimental.pallas import tpu as pltpu
from jax.experimental.pallas import tpu_sc as plsc

D = 32
B = 4096 * 200

NC = 2
NS = 16
NW = NC * NS
B_PER_W = B // NW
CHUNK = 1600
N_CHUNKS = B_PER_W // CHUNK


def _make_gather():
    mesh = plsc.VectorSubcoreMesh(core_axis_name="c", subcore_axis_name="s")

    @functools.partial(
        pl.kernel,
        mesh=mesh,
        compiler_params=pltpu.CompilerParams(use_tc_tiling_on_sc=False),
        out_type=jax.ShapeDtypeStruct((B, D), jnp.float32),
        scratch_types=[
            pltpu.VMEM((CHUNK,), jnp.int32),
            pltpu.VMEM((CHUNK, D), jnp.float32),
            pltpu.SemaphoreType.DMA,
        ],
    )
    def gather_k(idx_hbm, table_hbm, out_hbm, idx_v, rows_v, sem):
        wid = lax.axis_index("s") * NC + lax.axis_index("c")
        base = wid * B_PER_W

        def body(i, carry):
            off = base + i * CHUNK
            pltpu.sync_copy(idx_hbm.at[pl.ds(off, CHUNK)], idx_v)
            pltpu.async_copy(table_hbm.at[idx_v], rows_v, sem).wait()
            pltpu.sync_copy(rows_v, out_hbm.at[pl.ds(off, CHUNK)])
            return carry

        lax.fori_loop(0, N_CHUNKS, body, 0)

    return gather_k


_gather = _make_gather()


def kernel(input_variable, table):
    idx = input_variable.reshape(B).astype(jnp.int32)
    out = _gather(idx, table)
    return out.reshape(input_variable.shape[0], input_variable.shape[1], D)


# speedup vs baseline: 1.4979x; 1.0141x over previous
import functools

import jax
import jax.numpy as jnp
from jax import lax
from jax.experimental import pallas as pl
from jax.experimental.pallas import tpu as pltpu
from jax.experimental.pallas import tpu_sc as plsc

D = 32
B = 4096 * 200

NC = 2
NS = 16
NW = NC * NS
B_PER_W = B // NW
NBUF = 4
CHUNK = 800
N_CHUNKS = B_PER_W // CHUNK


def _make_gather():
    mesh = plsc.VectorSubcoreMesh(core_axis_name="c", subcore_axis_name="s")

    scratch = (
        [pltpu.VMEM((CHUNK,), jnp.int32) for _ in range(NBUF)]
        + [pltpu.VMEM((CHUNK, D), jnp.float32) for _ in range(NBUF)]
        + [pltpu.SemaphoreType.DMA for _ in range(2 * NBUF)]
    )

    @functools.partial(
        pl.kernel,
        mesh=mesh,
        compiler_params=pltpu.CompilerParams(use_tc_tiling_on_sc=False),
        out_type=jax.ShapeDtypeStruct((B, D), jnp.float32),
        scratch_types=scratch,
    )
    def gather_k(idx_hbm, table_hbm, out_hbm, *refs):
        idx_v = refs[:NBUF]
        rows_v = refs[NBUF:2 * NBUF]
        gsem = refs[2 * NBUF:3 * NBUF]
        osem = refs[3 * NBUF:4 * NBUF]

        wid = lax.axis_index("s") * NC + lax.axis_index("c")
        base = wid * B_PER_W

        gather_d = [None] * N_CHUNKS
        out_d = [None] * N_CHUNKS
        for t in range(N_CHUNKS + 1):
            if t < N_CHUNKS:
                b = t % NBUF
                if t >= NBUF:
                    out_d[t - NBUF].wait()
                off = base + t * CHUNK
                pltpu.sync_copy(idx_hbm.at[pl.ds(off, CHUNK)], idx_v[b])
                gather_d[t] = pltpu.async_copy(
                    table_hbm.at[idx_v[b]], rows_v[b], gsem[b])
            if t >= 1:
                g = t - 1
                b = g % NBUF
                gather_d[g].wait()
                off = base + g * CHUNK
                out_d[g] = pltpu.async_copy(
                    rows_v[b], out_hbm.at[pl.ds(off, CHUNK)], osem[b])
        for g in range(N_CHUNKS - NBUF, N_CHUNKS):
            out_d[g].wait()

    return gather_k


_gather = _make_gather()


def kernel(input_variable, table):
    idx = input_variable.reshape(B).astype(jnp.int32)
    out = _gather(idx, table)
    return out.reshape(input_variable.shape[0], input_variable.shape[1], D)
